# baseline (speedup 1.0000x reference)
import jax
import jax.numpy as jnp
import numpy as np
from jax.experimental import pallas as pl
from jax.experimental.pallas import tpu as pltpu

_KH, _KW, _ICH, _OCH = 3, 3, 32, 32
_P = 8
_L = _P * _ICH


def _make_w9_tables():
    idx = np.zeros((_KH, 3, _L, _L), np.int32)
    msk = np.zeros((_KH, 3, _L, _L), np.float32)
    ci = np.arange(_ICH)
    co = np.arange(_OCH)
    blk_idx = co[None, :] * (_KH * _KW * _ICH) + ci[:, None]
    for dh in range(_KH):
        for dw in range(_KW):
            for q in range(_P):
                p = q + dw - 1
                if p < 0:
                    p, dgi = p + _P, 0
                elif p >= _P:
                    p, dgi = p - _P, 2
                else:
                    dgi = 1
                sl = (dh, dgi, slice(p * _ICH, (p + 1) * _ICH),
                      slice(q * _ICH, (q + 1) * _ICH))
                idx[sl] = blk_idx + (dh * _KW + dw) * _ICH
                msk[sl] = 1.0
    return idx.reshape(9, _L, _L), msk.reshape(9, _L, _L)


_W9_IDX, _W9_MSK = _make_w9_tables()


def _build_w9(w):
    return jnp.take(w.reshape(-1), _W9_IDX) * _W9_MSK


def _conv_relu_chunk(load, w9_ref, bvec, ch, ng):
    acc = None
    for dh in range(3):
        for dgi in range(3):
            lhs = load(dh, dgi).reshape(ch * ng, _L)
            d = jnp.dot(lhs, w9_ref[3 * dh + dgi],
                        preferred_element_type=jnp.float32)
            acc = d if acc is None else acc + d
    return jnp.maximum(acc + bvec, 0.0).reshape(ch, ng, _L)


def _pool_repack(y):
    ch, ng = y.shape[0], y.shape[1]
    v = y.reshape(ch // 2, 2, ng, _L)
    y = jnp.maximum(v[:, 0], v[:, 1])
    r = pltpu.roll(y, _L - _ICH, axis=2)
    m = jnp.maximum(y, r)
    v = m.reshape(ch // 2, ng // 2, 2, _L)
    a, b = v[:, :, 0, :], v[:, :, 1, :]
    parts = [a[..., 0:32], a[..., 64:96], a[..., 128:160], a[..., 192:224],
             b[..., 0:32], b[..., 64:96], b[..., 128:160], b[..., 192:224]]
    return jnp.concatenate(parts, axis=-1)


def _make_net_kernel(h, g):
    h2, g2 = h // 2, g // 2

    def net_kernel(xp_ref, w9_ref, b_ref, out_ref, a1, a2, a3):
        bvec = b_ref[0]
        z1 = jnp.zeros((1, g + 2, _L), jnp.float32)
        z1c = jnp.zeros((h + 2, 1, _L), jnp.float32)
        a1[pl.ds(0, 1)] = z1
        a1[pl.ds(h + 1, 1)] = z1
        a1[:, pl.ds(0, 1), :] = z1c
        a1[:, pl.ds(g + 1, 1), :] = z1c
        z2 = jnp.zeros((1, g2 + 2, _L), jnp.float32)
        z2c = jnp.zeros((h2 + 2, 1, _L), jnp.float32)
        for ref in (a2, a3):
            ref[pl.ds(0, 1)] = z2
            ref[pl.ds(h2 + 1, 1)] = z2
            ref[:, pl.ds(0, 1), :] = z2c
            ref[:, pl.ds(g2 + 1, 1), :] = z2c

        def conv1(r, _):
            y = _conv_relu_chunk(
                lambda dh, dgi: xp_ref[0, pl.ds(r * 8 + dh, 8),
                                       pl.ds(dgi, g), :],
                w9_ref, bvec, 8, g)
            a1[pl.ds(1 + r * 8, 8), pl.ds(1, g), :] = y
            return 0

        def conv2_pool(r, _):
            y = _conv_relu_chunk(
                lambda dh, dgi: a1[pl.ds(r * 8 + dh, 8), pl.ds(dgi, g), :],
                w9_ref, bvec, 8, g)
            a2[pl.ds(1 + r * 4, 4), pl.ds(1, g2), :] = _pool_repack(y)
            return 0

        def conv3(r, _):
            y = _conv_relu_chunk(
                lambda dh, dgi: a2[pl.ds(r * 8 + dh, 8), pl.ds(dgi, g2), :],
                w9_ref, bvec, 8, g2)
            a3[pl.ds(1 + r * 8, 8), pl.ds(1, g2), :] = y
            return 0

        def conv4_pool(r, _):
            y = _conv_relu_chunk(
                lambda dh, dgi: a3[pl.ds(r * 8 + dh, 8), pl.ds(dgi, g2), :],
                w9_ref, bvec, 8, g2)
            out_ref[0, pl.ds(r * 4, 4), :, :] = _pool_repack(y)
            return 0

        jax.lax.fori_loop(0, h // 8, conv1, 0)
        jax.lax.fori_loop(0, h // 8, conv2_pool, 0)
        jax.lax.fori_loop(0, h2 // 8, conv3, 0)
        jax.lax.fori_loop(0, h2 // 8, conv4_pool, 0)

    return net_kernel


@jax.jit
def kernel(x, w, b):
    n, h, wd, _ = x.shape
    g = wd // _P
    h2, g2 = h // 2, g // 2
    h4, g4 = h // 4, g // 4
    w9 = _build_w9(w)
    xp = jnp.pad(x, ((0, 0), (1, 1), (_P, _P), (0, 0)))
    xp = xp.reshape(n, h + 2, g + 2, _L)
    bp = jnp.tile(b, _P).reshape(1, _L)

    out = pl.pallas_call(
        _make_net_kernel(h, g),
        grid=(n,),
        in_specs=[
            pl.BlockSpec((1, h + 2, g + 2, _L), lambda i: (i, 0, 0, 0)),
            pl.BlockSpec((9, _L, _L), lambda i: (0, 0, 0)),
            pl.BlockSpec((1, _L), lambda i: (0, 0)),
        ],
        out_specs=pl.BlockSpec((1, h4, g4, _L), lambda i: (i, 0, 0, 0)),
        out_shape=jax.ShapeDtypeStruct((n, h4, g4, _L), jnp.float32),
        scratch_shapes=[
            pltpu.VMEM((h + 2, g + 2, _L), jnp.float32),
            pltpu.VMEM((h2 + 2, g2 + 2, _L), jnp.float32),
            pltpu.VMEM((h2 + 2, g2 + 2, _L), jnp.float32),
        ],
        compiler_params=pltpu.CompilerParams(
            dimension_semantics=("parallel",),
            vmem_limit_bytes=100 * 1024 * 1024,
        ),
    )(xp, w9, bp)
    return out.reshape(n, h4, wd // 4, _OCH)


# speedup vs baseline: 4.2913x; 4.2913x over previous
import jax
import jax.numpy as jnp
import numpy as np
from jax.experimental import pallas as pl
from jax.experimental.pallas import tpu as pltpu

_KH, _KW, _ICH, _OCH = 3, 3, 32, 32
_P = 8
_L = _P * _ICH


def _make_sel():
    sel = np.zeros((3, _P, _P, _KW), np.float32)
    for dw in range(_KW):
        for q in range(_P):
            p = q + dw - 1
            if p < 0:
                p, dgi = p + _P, 0
            elif p >= _P:
                p, dgi = p - _P, 2
            else:
                dgi = 1
            sel[dgi, p, q, dw] = 1.0
    return sel


_SEL = _make_sel()


def _build_w9(w):
    w4 = w.reshape(_OCH, _KH, _KW, _ICH)
    w9 = jnp.einsum('gpqd,ohdc->hgpcqo', _SEL, w4)
    return w9.reshape(9, _L, _L)


def _conv_relu_chunk(load, w9_ref, bvec, ch, ng):
    acc = None
    for dh in range(3):
        for dgi in range(3):
            lhs = load(dh, dgi).reshape(ch * ng, _L)
            d = jnp.dot(lhs, w9_ref[3 * dh + dgi],
                        preferred_element_type=jnp.float32)
            acc = d if acc is None else acc + d
    return jnp.maximum(acc + bvec, 0.0).reshape(ch, ng, _L)


def _pool_repack(y):
    ch, ng = y.shape[0], y.shape[1]
    v = y.reshape(ch // 2, 2, ng, _L)
    y = jnp.maximum(v[:, 0], v[:, 1])
    r = pltpu.roll(y, _L - _ICH, axis=2)
    m = jnp.maximum(y, r)
    v = m.reshape(ch // 2, ng // 2, 2, _L)
    a, b = v[:, :, 0, :], v[:, :, 1, :]
    parts = [a[..., 0:32], a[..., 64:96], a[..., 128:160], a[..., 192:224],
             b[..., 0:32], b[..., 64:96], b[..., 128:160], b[..., 192:224]]
    return jnp.concatenate(parts, axis=-1)


def _make_net_kernel(h, g):
    h2, g2 = h // 2, g // 2

    def net_kernel(xp_ref, w9_ref, b_ref, out_ref, a1, a2, a3):
        bvec = b_ref[0]
        z1 = jnp.zeros((1, g + 2, _L), jnp.float32)
        z1c = jnp.zeros((h + 2, 1, _L), jnp.float32)
        a1[pl.ds(0, 1)] = z1
        a1[pl.ds(h + 1, 1)] = z1
        a1[:, pl.ds(0, 1), :] = z1c
        a1[:, pl.ds(g + 1, 1), :] = z1c
        z2 = jnp.zeros((1, g2 + 2, _L), jnp.float32)
        z2c = jnp.zeros((h2 + 2, 1, _L), jnp.float32)
        for ref in (a2, a3):
            ref[pl.ds(0, 1)] = z2
            ref[pl.ds(h2 + 1, 1)] = z2
            ref[:, pl.ds(0, 1), :] = z2c
            ref[:, pl.ds(g2 + 1, 1), :] = z2c

        def conv1(r, _):
            y = _conv_relu_chunk(
                lambda dh, dgi: xp_ref[0, pl.ds(r * 8 + dh, 8),
                                       pl.ds(dgi, g), :],
                w9_ref, bvec, 8, g)
            a1[pl.ds(1 + r * 8, 8), pl.ds(1, g), :] = y
            return 0

        def conv2_pool(r, _):
            y = _conv_relu_chunk(
                lambda dh, dgi: a1[pl.ds(r * 8 + dh, 8), pl.ds(dgi, g), :],
                w9_ref, bvec, 8, g)
            a2[pl.ds(1 + r * 4, 4), pl.ds(1, g2), :] = _pool_repack(y)
            return 0

        def conv3(r, _):
            y = _conv_relu_chunk(
                lambda dh, dgi: a2[pl.ds(r * 8 + dh, 8), pl.ds(dgi, g2), :],
                w9_ref, bvec, 8, g2)
            a3[pl.ds(1 + r * 8, 8), pl.ds(1, g2), :] = y
            return 0

        def conv4_pool(r, _):
            y = _conv_relu_chunk(
                lambda dh, dgi: a3[pl.ds(r * 8 + dh, 8), pl.ds(dgi, g2), :],
                w9_ref, bvec, 8, g2)
            out_ref[0, pl.ds(r * 4, 4), :, :] = _pool_repack(y)
            return 0

        jax.lax.fori_loop(0, h // 8, conv1, 0)
        jax.lax.fori_loop(0, h // 8, conv2_pool, 0)
        jax.lax.fori_loop(0, h2 // 8, conv3, 0)
        jax.lax.fori_loop(0, h2 // 8, conv4_pool, 0)

    return net_kernel


@jax.jit
def kernel(x, w, b):
    n, h, wd, _ = x.shape
    g = wd // _P
    h2, g2 = h // 2, g // 2
    h4, g4 = h // 4, g // 4
    w9 = _build_w9(w)
    xp = jnp.pad(x, ((0, 0), (1, 1), (_P, _P), (0, 0)))
    xp = xp.reshape(n, h + 2, g + 2, _L)
    bp = jnp.tile(b, _P).reshape(1, _L)

    out = pl.pallas_call(
        _make_net_kernel(h, g),
        grid=(n,),
        in_specs=[
            pl.BlockSpec((1, h + 2, g + 2, _L), lambda i: (i, 0, 0, 0)),
            pl.BlockSpec((9, _L, _L), lambda i: (0, 0, 0)),
            pl.BlockSpec((1, _L), lambda i: (0, 0)),
        ],
        out_specs=pl.BlockSpec((1, h4, g4, _L), lambda i: (i, 0, 0, 0)),
        out_shape=jax.ShapeDtypeStruct((n, h4, g4, _L), jnp.float32),
        scratch_shapes=[
            pltpu.VMEM((h + 2, g + 2, _L), jnp.float32),
            pltpu.VMEM((h2 + 2, g2 + 2, _L), jnp.float32),
            pltpu.VMEM((h2 + 2, g2 + 2, _L), jnp.float32),
        ],
        compiler_params=pltpu.CompilerParams(
            dimension_semantics=("parallel",),
            vmem_limit_bytes=100 * 1024 * 1024,
        ),
    )(xp, w9, bp)
    return out.reshape(n, h4, wd // 4, _OCH)


# speedup vs baseline: 4.9138x; 1.1450x over previous
import jax
import jax.numpy as jnp
import numpy as np
from jax.experimental import pallas as pl
from jax.experimental.pallas import tpu as pltpu

_KH, _KW, _ICH, _OCH = 3, 3, 32, 32
_P = 8
_L = _P * _ICH


def _make_sel():
    sel = np.zeros((3, _P, _P, _KW), np.float32)
    for dw in range(_KW):
        for q in range(_P):
            p = q + dw - 1
            if p < 0:
                p, dgi = p + _P, 0
            elif p >= _P:
                p, dgi = p - _P, 2
            else:
                dgi = 1
            sel[dgi, p, q, dw] = 1.0
    return sel


_SEL = _make_sel()


def _build_w9(w):
    w4 = w.reshape(_OCH, _KH, _KW, _ICH)
    w9 = jnp.einsum('gpqd,ohdc->hgpcqo', _SEL, w4)
    return w9.reshape(9, _L, _L)


def _conv_relu_chunk(load, w9_ref, bvec, ch, ng):
    acc = None
    for dh in range(3):
        for dgi in range(3):
            lhs = load(dh, dgi).reshape(ch * ng, _L)
            d = jnp.dot(lhs, w9_ref[3 * dh + dgi],
                        preferred_element_type=jnp.float32)
            acc = d if acc is None else acc + d
    return jnp.maximum(acc + bvec, 0.0).reshape(ch, ng, _L)


def _pool_repack(y):
    ch, ng = y.shape[0], y.shape[1]
    v = y.reshape(ch // 2, 2, ng, _L)
    y = jnp.maximum(v[:, 0], v[:, 1])
    r = pltpu.roll(y, _L - _ICH, axis=2)
    m = jnp.maximum(y, r)
    v = m.reshape(ch // 2, ng // 2, 2, _L)
    a, b = v[:, :, 0, :], v[:, :, 1, :]
    parts = [a[..., 0:32], a[..., 64:96], a[..., 128:160], a[..., 192:224],
             b[..., 0:32], b[..., 64:96], b[..., 128:160], b[..., 192:224]]
    return jnp.concatenate(parts, axis=-1)


def _make_net_kernel(h, g):
    h2, g2 = h // 2, g // 2

    def net_kernel(xr_ref, w9_ref, b_ref, out_ref, a0, a1, a2, a3):
        bvec = b_ref[0]
        z1 = jnp.zeros((1, g + 2, _L), jnp.float32)
        z1c = jnp.zeros((h + 2, 1, _L), jnp.float32)
        for ref in (a0, a1):
            ref[pl.ds(0, 1)] = z1
            ref[pl.ds(h + 1, 1)] = z1
            ref[:, pl.ds(0, 1), :] = z1c
            ref[:, pl.ds(g + 1, 1), :] = z1c
        z2 = jnp.zeros((1, g2 + 2, _L), jnp.float32)
        z2c = jnp.zeros((h2 + 2, 1, _L), jnp.float32)
        for ref in (a2, a3):
            ref[pl.ds(0, 1)] = z2
            ref[pl.ds(h2 + 1, 1)] = z2
            ref[:, pl.ds(0, 1), :] = z2c
            ref[:, pl.ds(g2 + 1, 1), :] = z2c

        def stage(r, _):
            a0[pl.ds(1 + r * 8, 8), pl.ds(1, g), :] = \
                xr_ref[0, pl.ds(r * 8, 8), :, :]
            return 0

        def conv1(r, _):
            y = _conv_relu_chunk(
                lambda dh, dgi: a0[pl.ds(r * 8 + dh, 8), pl.ds(dgi, g), :],
                w9_ref, bvec, 8, g)
            a1[pl.ds(1 + r * 8, 8), pl.ds(1, g), :] = y
            return 0

        def conv2_pool(r, _):
            y = _conv_relu_chunk(
                lambda dh, dgi: a1[pl.ds(r * 8 + dh, 8), pl.ds(dgi, g), :],
                w9_ref, bvec, 8, g)
            a2[pl.ds(1 + r * 4, 4), pl.ds(1, g2), :] = _pool_repack(y)
            return 0

        def conv3(r, _):
            y = _conv_relu_chunk(
                lambda dh, dgi: a2[pl.ds(r * 8 + dh, 8), pl.ds(dgi, g2), :],
                w9_ref, bvec, 8, g2)
            a3[pl.ds(1 + r * 8, 8), pl.ds(1, g2), :] = y
            return 0

        def conv4_pool(r, _):
            y = _conv_relu_chunk(
                lambda dh, dgi: a3[pl.ds(r * 8 + dh, 8), pl.ds(dgi, g2), :],
                w9_ref, bvec, 8, g2)
            out_ref[0, pl.ds(r * 4, 4), :, :] = _pool_repack(y)
            return 0

        jax.lax.fori_loop(0, h // 8, stage, 0)
        jax.lax.fori_loop(0, h // 8, conv1, 0)
        jax.lax.fori_loop(0, h // 8, conv2_pool, 0)
        jax.lax.fori_loop(0, h2 // 8, conv3, 0)
        jax.lax.fori_loop(0, h2 // 8, conv4_pool, 0)

    return net_kernel


@jax.jit
def kernel(x, w, b):
    n, h, wd, _ = x.shape
    g = wd // _P
    h2, g2 = h // 2, g // 2
    h4, g4 = h // 4, g // 4
    w9 = _build_w9(w)
    xr = x.reshape(n, h, g, _L)
    bp = jnp.tile(b, _P).reshape(1, _L)

    out = pl.pallas_call(
        _make_net_kernel(h, g),
        grid=(n,),
        in_specs=[
            pl.BlockSpec((1, h, g, _L), lambda i: (i, 0, 0, 0)),
            pl.BlockSpec((9, _L, _L), lambda i: (0, 0, 0)),
            pl.BlockSpec((1, _L), lambda i: (0, 0)),
        ],
        out_specs=pl.BlockSpec((1, h4, g4, _L), lambda i: (i, 0, 0, 0)),
        out_shape=jax.ShapeDtypeStruct((n, h4, g4, _L), jnp.float32),
        scratch_shapes=[
            pltpu.VMEM((h + 2, g + 2, _L), jnp.float32),
            pltpu.VMEM((h + 2, g + 2, _L), jnp.float32),
            pltpu.VMEM((h2 + 2, g2 + 2, _L), jnp.float32),
            pltpu.VMEM((h2 + 2, g2 + 2, _L), jnp.float32),
        ],
        compiler_params=pltpu.CompilerParams(
            dimension_semantics=("parallel",),
            vmem_limit_bytes=100 * 1024 * 1024,
        ),
    )(xr, w9, bp)
    return out.reshape(n, h4, wd // 4, _OCH)


# speedup vs baseline: 4.9247x; 1.0022x over previous
import jax
import jax.numpy as jnp
import numpy as np
from jax.experimental import pallas as pl
from jax.experimental.pallas import tpu as pltpu

_KH, _KW, _ICH, _OCH = 3, 3, 32, 32
_P = 8
_L = _P * _ICH


def _make_sel():
    sel = np.zeros((3, _P, _P, _KW), np.float32)
    for dw in range(_KW):
        for q in range(_P):
            p = q + dw - 1
            if p < 0:
                p, dgi = p + _P, 0
            elif p >= _P:
                p, dgi = p - _P, 2
            else:
                dgi = 1
            sel[dgi, p, q, dw] = 1.0
    return sel


_SEL = _make_sel()


def _build_w9(w):
    w4 = w.reshape(_OCH, _KH, _KW, _ICH)
    w9 = jnp.einsum('gpqd,ohdc->hgpcqo', _SEL, w4)
    return w9.reshape(9, _L, _L)


def _conv_relu_chunk(load, w9_ref, bvec, ch, ng):
    acc = None
    for dh in range(3):
        for dgi in range(3):
            lhs = load(dh, dgi).reshape(ch * ng, _L)
            d = jnp.dot(lhs, w9_ref[3 * dh + dgi],
                        preferred_element_type=jnp.float32)
            acc = d if acc is None else acc + d
    return jnp.maximum(acc + bvec, 0.0).reshape(ch, ng, _L)


def _pool_repack(y):
    ch, ng = y.shape[0], y.shape[1]
    v = y.reshape(ch // 2, 2, ng, _L)
    y = jnp.maximum(v[:, 0], v[:, 1])
    r = pltpu.roll(y, _L - _ICH, axis=2)
    m = jnp.maximum(y, r)
    v = m.reshape(ch // 2, ng // 2, 2, _L)
    a, b = v[:, :, 0, :], v[:, :, 1, :]
    parts = [a[..., 0:32], a[..., 64:96], a[..., 128:160], a[..., 192:224],
             b[..., 0:32], b[..., 64:96], b[..., 128:160], b[..., 192:224]]
    return jnp.concatenate(parts, axis=-1)


def _make_net_kernel(h, g):
    h2, g2 = h // 2, g // 2

    def net_kernel(xr_ref, w9_ref, b_ref, out_ref, a0, a1, a2, a3):
        bvec = b_ref[0]
        z1 = jnp.zeros((1, g + 2, _L), jnp.float32)
        z1c = jnp.zeros((h + 2, 1, _L), jnp.float32)
        for ref in (a0, a1):
            ref[pl.ds(0, 1)] = z1
            ref[pl.ds(h + 1, 1)] = z1
            ref[:, pl.ds(0, 1), :] = z1c
            ref[:, pl.ds(g + 1, 1), :] = z1c
        z2 = jnp.zeros((1, g2 + 2, _L), jnp.float32)
        z2c = jnp.zeros((h2 + 2, 1, _L), jnp.float32)
        for ref in (a2, a3):
            ref[pl.ds(0, 1)] = z2
            ref[pl.ds(h2 + 1, 1)] = z2
            ref[:, pl.ds(0, 1), :] = z2c
            ref[:, pl.ds(g2 + 1, 1), :] = z2c

        def stage(r, _):
            a0[pl.ds(1 + r * 8, 8), pl.ds(1, g), :] = \
                xr_ref[0, pl.ds(r * 8, 8), :, :]
            return 0

        def conv1(r, _):
            y = _conv_relu_chunk(
                lambda dh, dgi: a0[pl.ds(r * 8 + dh, 8), pl.ds(dgi, g), :],
                w9_ref, bvec, 8, g)
            a1[pl.ds(1 + r * 8, 8), pl.ds(1, g), :] = y
            return 0

        def conv2_pool(r, _):
            y = _conv_relu_chunk(
                lambda dh, dgi: a1[pl.ds(r * 8 + dh, 8), pl.ds(dgi, g), :],
                w9_ref, bvec, 8, g)
            a2[pl.ds(1 + r * 4, 4), pl.ds(1, g2), :] = _pool_repack(y)
            return 0

        def conv3(r, _):
            y = _conv_relu_chunk(
                lambda dh, dgi: a2[pl.ds(r * 8 + dh, 8), pl.ds(dgi, g2), :],
                w9_ref, bvec, 8, g2)
            a3[pl.ds(1 + r * 8, 8), pl.ds(1, g2), :] = y
            return 0

        def conv4_pool(r, _):
            y = _conv_relu_chunk(
                lambda dh, dgi: a3[pl.ds(r * 8 + dh, 8), pl.ds(dgi, g2), :],
                w9_ref, bvec, 8, g2)
            out_ref[0, pl.ds(r * 4, 4), :, :] = _pool_repack(y)
            return 0

        jax.lax.fori_loop(0, h // 8, stage, 0)
        jax.lax.fori_loop(0, h // 8, conv1, 0)
        jax.lax.fori_loop(0, h // 8, conv2_pool, 0)
        jax.lax.fori_loop(0, h2 // 8, conv3, 0)
        jax.lax.fori_loop(0, h2 // 8, conv4_pool, 0)

    return net_kernel


@jax.jit
def kernel(x, w, b):
    n, h, wd, _ = x.shape
    g = wd // _P
    h2, g2 = h // 2, g // 2
    h4, g4 = h // 4, g // 4
    w9 = _build_w9(w)
    xr = x.reshape(n, h, g, _L)
    bp = jnp.tile(b, _P).reshape(1, _L)

    out = pl.pallas_call(
        _make_net_kernel(h, g),
        grid=(n,),
        in_specs=[
            pl.BlockSpec((1, h, g, _L), lambda i: (i, 0, 0, 0)),
            pl.BlockSpec((9, _L, _L), lambda i: (0, 0, 0)),
            pl.BlockSpec((1, _L), lambda i: (0, 0)),
        ],
        out_specs=pl.BlockSpec((1, h4, g4, _L), lambda i: (i, 0, 0, 0)),
        out_shape=jax.ShapeDtypeStruct((n, h4, g4, _L), jnp.float32),
        scratch_shapes=[
            pltpu.VMEM((h + 2, g + 2, _L), jnp.float32),
            pltpu.VMEM((h + 2, g + 2, _L), jnp.float32),
            pltpu.VMEM((h2 + 2, g2 + 2, _L), jnp.float32),
            pltpu.VMEM((h2 + 2, g2 + 2, _L), jnp.float32),
        ],
        compiler_params=pltpu.CompilerParams(
            dimension_semantics=("parallel",),
            allow_input_fusion=[True, False, False],
            vmem_limit_bytes=100 * 1024 * 1024,
        ),
    )(xr, w9, bp)
    return out.reshape(n, h4, wd // 4, _OCH)


# speedup vs baseline: 6.0959x; 1.2378x over previous
import jax
import jax.numpy as jnp
import numpy as np
from jax.experimental import pallas as pl
from jax.experimental.pallas import tpu as pltpu

_KH, _KW, _ICH, _OCH = 3, 3, 32, 32
_P = 8
_L = _P * _ICH


def _make_sel():
    sel = np.zeros((3, _P, _P, _KW), np.float32)
    for dw in range(_KW):
        for q in range(_P):
            p = q + dw - 1
            if p < 0:
                p, dgi = p + _P, 0
            elif p >= _P:
                p, dgi = p - _P, 2
            else:
                dgi = 1
            sel[dgi, p, q, dw] = 1.0
    return sel


_SEL = _make_sel()


def _build_w9(w):
    w4 = w.reshape(_OCH, _KH, _KW, _ICH)
    w9 = jnp.einsum('gpqd,ohdc->hgpcqo', _SEL, w4)
    return w9.reshape(9, _L, _L)


def _conv_relu_chunk(load, w9_ref, bvec, ch, ng):
    acc = None
    for dh in range(3):
        for dgi in range(3):
            lhs = load(dh, dgi).reshape(ch * ng, _L)
            d = jnp.dot(lhs, w9_ref[3 * dh + dgi],
                        preferred_element_type=jnp.float32)
            acc = d if acc is None else acc + d
    return jnp.maximum(acc + bvec, 0.0).reshape(ch, ng, _L)


def _pool_repack(y):
    ch, ng = y.shape[0], y.shape[1]
    v = y.reshape(ch // 2, 2, ng, _L)
    y = jnp.maximum(v[:, 0], v[:, 1])
    r = pltpu.roll(y, _L - _ICH, axis=2)
    m = jnp.maximum(y, r)
    v = m.reshape(ch // 2, ng // 2, 2, _L)
    a, b = v[:, :, 0, :], v[:, :, 1, :]
    parts = [a[..., 0:32], a[..., 64:96], a[..., 128:160], a[..., 192:224],
             b[..., 0:32], b[..., 64:96], b[..., 128:160], b[..., 192:224]]
    return jnp.concatenate(parts, axis=-1)


def _make_net_kernel(h, g):
    h2, g2 = h // 2, g // 2

    def net_kernel(xr_ref, w9_ref, b_ref, out_ref, a0, a1, a2, a3):
        bvec = b_ref[0]
        z1 = jnp.zeros((1, g + 2, _L), jnp.float32)
        z1c = jnp.zeros((h + 2, 1, _L), jnp.float32)
        for ref in (a0, a1):
            ref[pl.ds(0, 1)] = z1
            ref[pl.ds(h + 1, 1)] = z1
            ref[:, pl.ds(0, 1), :] = z1c
            ref[:, pl.ds(g + 1, 1), :] = z1c
        z2 = jnp.zeros((1, g2 + 2, _L), jnp.float32)
        z2c = jnp.zeros((h2 + 2, 1, _L), jnp.float32)
        for ref in (a2, a3):
            ref[pl.ds(0, 1)] = z2
            ref[pl.ds(h2 + 1, 1)] = z2
            ref[:, pl.ds(0, 1), :] = z2c
            ref[:, pl.ds(g2 + 1, 1), :] = z2c

        def stage(r, _):
            a0[pl.ds(1 + r * 8, 8), pl.ds(1, g), :] = \
                xr_ref[0, pl.ds(r * 8, 8), :, :]
            return 0

        def conv1(r2, _):
            for u in range(2):
                r = r2 * 2 + u
                y = _conv_relu_chunk(
                    lambda dh, dgi: a0[pl.ds(r * 8 + dh, 8),
                                       pl.ds(dgi, g), :],
                    w9_ref, bvec, 8, g)
                a1[pl.ds(1 + r * 8, 8), pl.ds(1, g), :] = y
            return 0

        def conv2_pool(r2, _):
            for u in range(2):
                r = r2 * 2 + u
                y = _conv_relu_chunk(
                    lambda dh, dgi: a1[pl.ds(r * 8 + dh, 8),
                                       pl.ds(dgi, g), :],
                    w9_ref, bvec, 8, g)
                a2[pl.ds(1 + r * 4, 4), pl.ds(1, g2), :] = _pool_repack(y)
            return 0

        def conv3(r2, _):
            for u in range(2):
                r = r2 * 2 + u
                y = _conv_relu_chunk(
                    lambda dh, dgi: a2[pl.ds(r * 8 + dh, 8),
                                       pl.ds(dgi, g2), :],
                    w9_ref, bvec, 8, g2)
                a3[pl.ds(1 + r * 8, 8), pl.ds(1, g2), :] = y
            return 0

        def conv4_pool(r2, _):
            for u in range(2):
                r = r2 * 2 + u
                y = _conv_relu_chunk(
                    lambda dh, dgi: a3[pl.ds(r * 8 + dh, 8),
                                       pl.ds(dgi, g2), :],
                    w9_ref, bvec, 8, g2)
                out_ref[0, pl.ds(r * 4, 4), :, :] = _pool_repack(y)
            return 0

        jax.lax.fori_loop(0, h // 8, stage, 0)
        jax.lax.fori_loop(0, h // 16, conv1, 0)
        jax.lax.fori_loop(0, h // 16, conv2_pool, 0)
        jax.lax.fori_loop(0, h2 // 16, conv3, 0)
        jax.lax.fori_loop(0, h2 // 16, conv4_pool, 0)

    return net_kernel


@jax.jit
def kernel(x, w, b):
    n, h, wd, _ = x.shape
    g = wd // _P
    h2, g2 = h // 2, g // 2
    h4, g4 = h // 4, g // 4
    w9 = _build_w9(w)
    xr = x.reshape(n, h, g, _L)
    bp = jnp.tile(b, _P).reshape(1, _L)

    out = pl.pallas_call(
        _make_net_kernel(h, g),
        grid=(n,),
        in_specs=[
            pl.BlockSpec((1, h, g, _L), lambda i: (i, 0, 0, 0)),
            pl.BlockSpec((9, _L, _L), lambda i: (0, 0, 0)),
            pl.BlockSpec((1, _L), lambda i: (0, 0)),
        ],
        out_specs=pl.BlockSpec((1, h4, g4, _L), lambda i: (i, 0, 0, 0)),
        out_shape=jax.ShapeDtypeStruct((n, h4, g4, _L), jnp.float32),
        scratch_shapes=[
            pltpu.VMEM((h + 2, g + 2, _L), jnp.float32),
            pltpu.VMEM((h + 2, g + 2, _L), jnp.float32),
            pltpu.VMEM((h2 + 2, g2 + 2, _L), jnp.float32),
            pltpu.VMEM((h2 + 2, g2 + 2, _L), jnp.float32),
        ],
        compiler_params=pltpu.CompilerParams(
            dimension_semantics=("parallel",),
            allow_input_fusion=[True, False, False],
            vmem_limit_bytes=100 * 1024 * 1024,
        ),
    )(xr, w9, bp)
    return out.reshape(n, h4, wd // 4, _OCH)


# speedup vs baseline: 6.8742x; 1.1277x over previous
import jax
import jax.numpy as jnp
import numpy as np
from jax.experimental import pallas as pl
from jax.experimental.pallas import tpu as pltpu

_KH, _KW, _ICH, _OCH = 3, 3, 32, 32
_P = 8
_L = _P * _ICH


def _make_sel():
    sel = np.zeros((3, _P, _P, _KW), np.float32)
    for dw in range(_KW):
        for q in range(_P):
            p = q + dw - 1
            if p < 0:
                p, dgi = p + _P, 0
            elif p >= _P:
                p, dgi = p - _P, 2
            else:
                dgi = 1
            sel[dgi, p, q, dw] = 1.0
    return sel


_SEL = _make_sel()


def _build_w9(w):
    w4 = w.reshape(_OCH, _KH, _KW, _ICH)
    w9 = jnp.einsum('gpqd,ohdc->hgpcqo', _SEL, w4)
    return w9.reshape(9, _L, _L)


def _conv_relu_chunk(load, w9_ref, bvec, ch, ng):
    acc = None
    for dh in range(3):
        for dgi in range(3):
            lhs = load(dh, dgi).reshape(ch * ng, _L)
            d = jnp.dot(lhs, w9_ref[3 * dh + dgi],
                        preferred_element_type=jnp.float32)
            acc = d if acc is None else acc + d
    return jnp.maximum(acc + bvec, 0.0).reshape(ch, ng, _L)


def _pool_repack(y):
    ch, ng = y.shape[0], y.shape[1]
    v = y.reshape(ch // 2, 2, ng, _L)
    y = jnp.maximum(v[:, 0], v[:, 1])
    r = pltpu.roll(y, _L - _ICH, axis=2)
    m = jnp.maximum(y, r)
    v = m.reshape(ch // 2, ng // 2, 2, _L)
    a, b = v[:, :, 0, :], v[:, :, 1, :]
    parts = [a[..., 0:32], a[..., 64:96], a[..., 128:160], a[..., 192:224],
             b[..., 0:32], b[..., 64:96], b[..., 128:160], b[..., 192:224]]
    return jnp.concatenate(parts, axis=-1)


def _make_net_kernel(h, g):
    h2, g2 = h // 2, g // 2

    def net_kernel(xr_ref, w9_ref, b_ref, out_ref, a0, a1, a2, a3):
        bvec = b_ref[0]
        z1 = jnp.zeros((1, g + 2, _L), jnp.float32)
        z1c = jnp.zeros((h + 2, 1, _L), jnp.float32)
        for ref in (a0, a1):
            ref[pl.ds(0, 1)] = z1
            ref[pl.ds(h + 1, 1)] = z1
            ref[:, pl.ds(0, 1), :] = z1c
            ref[:, pl.ds(g + 1, 1), :] = z1c
        z2 = jnp.zeros((1, g2 + 2, _L), jnp.float32)
        z2c = jnp.zeros((h2 + 2, 1, _L), jnp.float32)
        for ref in (a2, a3):
            ref[pl.ds(0, 1)] = z2
            ref[pl.ds(h2 + 1, 1)] = z2
            ref[:, pl.ds(0, 1), :] = z2c
            ref[:, pl.ds(g2 + 1, 1), :] = z2c

        def stage(r, _):
            a0[pl.ds(1 + r * 8, 8), pl.ds(1, g), :] = \
                xr_ref[0, pl.ds(r * 8, 8), :, :]
            return 0

        def conv1(r2, _):
            for u in range(4):
                r = r2 * 4 + u
                y = _conv_relu_chunk(
                    lambda dh, dgi: a0[pl.ds(r * 8 + dh, 8),
                                       pl.ds(dgi, g), :],
                    w9_ref, bvec, 8, g)
                a1[pl.ds(1 + r * 8, 8), pl.ds(1, g), :] = y
            return 0

        def conv2_pool(r2, _):
            for u in range(4):
                r = r2 * 4 + u
                y = _conv_relu_chunk(
                    lambda dh, dgi: a1[pl.ds(r * 8 + dh, 8),
                                       pl.ds(dgi, g), :],
                    w9_ref, bvec, 8, g)
                a2[pl.ds(1 + r * 4, 4), pl.ds(1, g2), :] = _pool_repack(y)
            return 0

        def conv3(r2, _):
            for u in range(4):
                r = r2 * 4 + u
                y = _conv_relu_chunk(
                    lambda dh, dgi: a2[pl.ds(r * 8 + dh, 8),
                                       pl.ds(dgi, g2), :],
                    w9_ref, bvec, 8, g2)
                a3[pl.ds(1 + r * 8, 8), pl.ds(1, g2), :] = y
            return 0

        def conv4_pool(r2, _):
            for u in range(4):
                r = r2 * 4 + u
                y = _conv_relu_chunk(
                    lambda dh, dgi: a3[pl.ds(r * 8 + dh, 8),
                                       pl.ds(dgi, g2), :],
                    w9_ref, bvec, 8, g2)
                out_ref[0, pl.ds(r * 4, 4), :, :] = _pool_repack(y)
            return 0

        jax.lax.fori_loop(0, h // 8, stage, 0)
        jax.lax.fori_loop(0, h // 32, conv1, 0)
        jax.lax.fori_loop(0, h // 32, conv2_pool, 0)
        jax.lax.fori_loop(0, h2 // 32, conv3, 0)
        jax.lax.fori_loop(0, h2 // 32, conv4_pool, 0)

    return net_kernel


@jax.jit
def kernel(x, w, b):
    n, h, wd, _ = x.shape
    g = wd // _P
    h2, g2 = h // 2, g // 2
    h4, g4 = h // 4, g // 4
    w9 = _build_w9(w)
    xr = x.reshape(n, h, g, _L)
    bp = jnp.tile(b, _P).reshape(1, _L)

    out = pl.pallas_call(
        _make_net_kernel(h, g),
        grid=(n,),
        in_specs=[
            pl.BlockSpec((1, h, g, _L), lambda i: (i, 0, 0, 0)),
            pl.BlockSpec((9, _L, _L), lambda i: (0, 0, 0)),
            pl.BlockSpec((1, _L), lambda i: (0, 0)),
        ],
        out_specs=pl.BlockSpec((1, h4, g4, _L), lambda i: (i, 0, 0, 0)),
        out_shape=jax.ShapeDtypeStruct((n, h4, g4, _L), jnp.float32),
        scratch_shapes=[
            pltpu.VMEM((h + 2, g + 2, _L), jnp.float32),
            pltpu.VMEM((h + 2, g + 2, _L), jnp.float32),
            pltpu.VMEM((h2 + 2, g2 + 2, _L), jnp.float32),
            pltpu.VMEM((h2 + 2, g2 + 2, _L), jnp.float32),
        ],
        compiler_params=pltpu.CompilerParams(
            dimension_semantics=("parallel",),
            allow_input_fusion=[True, False, False],
            vmem_limit_bytes=100 * 1024 * 1024,
        ),
    )(xr, w9, bp)
    return out.reshape(n, h4, wd // 4, _OCH)
